# baseline (speedup 1.0000x reference)
import functools

import jax
import jax.numpy as jnp
from jax import lax
from jax.experimental import pallas as pl
from jax.experimental.pallas import tpu as pltpu
from jax.experimental.pallas import tpu_sc as plsc

NC = 2
NS = 16
NW = NC * NS
CH = 128


@functools.partial(jax.jit, static_argnames=("n_rows", "dim", "vocab"))
def _sc_gather(idx2d, table, n_rows, dim, vocab):
    b_per_w = n_rows // NW
    n_chunks = b_per_w // CH
    v_main = (vocab // (8 * NS)) * 8
    v_rem = vocab - v_main * NS

    def body(table_hbm, idx_hbm, out_hbm, shared_tab,
             ib0, ib1, rb0, rb1, gsem0, gsem1, isem0, isem1, wsem0, wsem1):
        cid = lax.axis_index("c")
        sid = lax.axis_index("s")
        wid = sid * NC + cid
        base = wid * b_per_w

        pltpu.sync_copy(
            table_hbm.at[pl.ds(sid * v_main, v_main)],
            shared_tab.at[pl.ds(sid * v_main, v_main)],
        )
        if v_rem:
            @pl.when(sid == 0)
            def _():
                pltpu.sync_copy(
                    table_hbm.at[pl.ds(NS * v_main, v_rem)],
                    shared_tab.at[pl.ds(NS * v_main, v_rem)],
                )
        plsc.subcore_barrier()

        ibs = (ib0, ib1)
        rbs = (rb0, rb1)
        gsems = (gsem0, gsem1)
        isems = (isem0, isem1)
        wsems = (wsem0, wsem1)
        chunk0 = wid * n_chunks

        pltpu.sync_copy(idx_hbm.at[pl.ds(chunk0, 1)], ib0)
        pltpu.sync_copy(idx_hbm.at[pl.ds(chunk0 + 1, 1)], ib1)
        pltpu.async_copy(shared_tab.at[ib0.at[0]], rb0, gsem0)

        @pl.loop(0, n_chunks, step=2)
        def _(j):
            for b in range(2):
                b1 = 1 - b
                jj = j + b
                pltpu.make_async_copy(shared_tab.at[ibs[b].at[0]], rbs[b],
                                      gsems[b]).wait()
                out_slc = out_hbm.at[pl.ds(base + jj * CH, CH)]
                pltpu.async_copy(rbs[b], out_slc, wsems[b])

                @pl.when(jj + 2 < n_chunks)
                def _():
                    pltpu.async_copy(idx_hbm.at[pl.ds(chunk0 + jj + 2, 1)],
                                     ibs[b], isems[b])

                @pl.when(jj + 1 < n_chunks)
                def _():
                    @pl.when(jj >= 1)
                    def _():
                        prev = out_hbm.at[pl.ds(base + (jj - 1) * CH, CH)]
                        pltpu.make_async_copy(rbs[b1], prev, wsems[b1]).wait()
                        pltpu.make_async_copy(
                            idx_hbm.at[pl.ds(chunk0 + jj + 1, 1)],
                            ibs[b1], isems[b1]).wait()
                    pltpu.async_copy(shared_tab.at[ibs[b1].at[0]], rbs[b1],
                                     gsems[b1])

        pltpu.make_async_copy(
            rb0, out_hbm.at[pl.ds(base + (n_chunks - 2) * CH, CH)], wsem0).wait()
        pltpu.make_async_copy(
            rb1, out_hbm.at[pl.ds(base + (n_chunks - 1) * CH, CH)], wsem1).wait()

    mesh = plsc.VectorSubcoreMesh(core_axis_name="c", subcore_axis_name="s")
    f = pl.kernel(
        body,
        out_type=jax.ShapeDtypeStruct((n_rows, dim), jnp.float32),
        mesh=mesh,
        scratch_types=[
            pltpu.VMEM_SHARED((vocab, dim), jnp.float32),
            pltpu.VMEM((1, CH), jnp.int32),
            pltpu.VMEM((1, CH), jnp.int32),
            pltpu.VMEM((CH, dim), jnp.float32),
            pltpu.VMEM((CH, dim), jnp.float32),
            pltpu.SemaphoreType.DMA,
            pltpu.SemaphoreType.DMA,
            pltpu.SemaphoreType.DMA,
            pltpu.SemaphoreType.DMA,
            pltpu.SemaphoreType.DMA,
            pltpu.SemaphoreType.DMA,
        ],
    )
    return f(table, idx2d)


def kernel(x, embedding):
    b, h = x.shape
    v, d = embedding.shape
    n_rows = b * h
    assert n_rows % (NW * CH * 2) == 0
    idx2d = x.reshape(n_rows // CH, CH)
    out = _sc_gather(idx2d, embedding, n_rows, d, v)
    return out.reshape(b, h, d)


# speedup vs baseline: 1.0365x; 1.0365x over previous
import functools

import jax
import jax.numpy as jnp
from jax import lax
from jax.experimental import pallas as pl
from jax.experimental.pallas import tpu as pltpu
from jax.experimental.pallas import tpu_sc as plsc

NC = 2
NS = 16
NW = NC * NS
CH = 128


@functools.partial(jax.jit, static_argnames=("n_rows", "dim", "vocab"))
def _sc_gather(idx2d, table, n_rows, dim, vocab):
    b_per_w = n_rows // NW
    n_chunks = b_per_w // CH
    v_main = (vocab // (8 * NS)) * 8
    v_rem = vocab - v_main * NS

    def body(table_hbm, idx_hbm, out_hbm, shared_tab,
             idx_v, rb0, rb1, gsem0, gsem1):
        cid = lax.axis_index("c")
        sid = lax.axis_index("s")
        wid = sid * NC + cid
        base = wid * b_per_w

        pltpu.sync_copy(
            table_hbm.at[pl.ds(sid * v_main, v_main)],
            shared_tab.at[pl.ds(sid * v_main, v_main)],
        )
        if v_rem:
            @pl.when(sid == 0)
            def _():
                pltpu.sync_copy(
                    table_hbm.at[pl.ds(NS * v_main, v_rem)],
                    shared_tab.at[pl.ds(NS * v_main, v_rem)],
                )
        plsc.subcore_barrier()

        rbs = (rb0, rb1)
        gsems = (gsem0, gsem1)
        chunk0 = wid * n_chunks

        ph0 = (n_chunks // 2) // 8 * 8
        for p0, pn in ((0, ph0), (ph0, n_chunks - ph0)):
            pltpu.sync_copy(idx_hbm.at[pl.ds(chunk0 + p0, pn)],
                            idx_v.at[pl.ds(0, pn)])
            pltpu.async_copy(shared_tab.at[idx_v.at[0]], rb0, gsem0)
            pltpu.async_copy(shared_tab.at[idx_v.at[1]], rb1, gsem1)

            @pl.loop(0, pn, step=2)
            def _(j):
                for b in range(2):
                    jj = j + b
                    rb, gsem = rbs[b], gsems[b]
                    pltpu.make_async_copy(shared_tab.at[idx_v.at[jj]], rb,
                                          gsem).wait()
                    pltpu.sync_copy(
                        rb, out_hbm.at[pl.ds(base + (p0 + jj) * CH, CH)])

                    @pl.when(jj + 2 < pn)
                    def _():
                        pltpu.async_copy(shared_tab.at[idx_v.at[jj + 2]], rb,
                                         gsem)

    mesh = plsc.VectorSubcoreMesh(core_axis_name="c", subcore_axis_name="s")
    f = pl.kernel(
        body,
        out_type=jax.ShapeDtypeStruct((n_rows, dim), jnp.float32),
        mesh=mesh,
        scratch_types=[
            pltpu.VMEM_SHARED((vocab, dim), jnp.float32),
            pltpu.VMEM(((n_chunks // 2) // 8 * 8 + 8, CH), jnp.int32),
            pltpu.VMEM((CH, dim), jnp.float32),
            pltpu.VMEM((CH, dim), jnp.float32),
            pltpu.SemaphoreType.DMA,
            pltpu.SemaphoreType.DMA,
        ],
    )
    return f(table, idx2d)


def kernel(x, embedding):
    b, h = x.shape
    v, d = embedding.shape
    n_rows = b * h
    assert n_rows % (NW * CH * 2) == 0
    idx2d = x.reshape(n_rows // CH, CH)
    out = _sc_gather(idx2d, embedding, n_rows, d, v)
    return out.reshape(b, h, d)
